# baseline (speedup 1.0000x reference)
import functools

import jax
import jax.numpy as jnp
from jax import lax
from jax.experimental import pallas as pl
from jax.experimental.pallas import tpu as pltpu
from jax.experimental.pallas import tpu_sc as plsc

_N = 10000
_E = 320000
_D = 128

_F32 = jnp.float32


_NB = 1000
_EB = 2000


def _dot(a, b):
    return jnp.dot(a, b, preferred_element_type=_F32)


def _node_mlp_body(x_ref, w1, b1, w2, b2, w3, b3, o_ref):
    h = jnp.maximum(_dot(x_ref[...], w1[...]) + b1[...], 0.0)
    h = jnp.maximum(_dot(h, w2[...]) + b2[...], 0.0)
    o_ref[...] = _dot(h, w3[...]) + b3[...]


def _full(shape):
    return pl.BlockSpec(shape, lambda i: (0,) * len(shape))


def _node_mlp(x, p):
    w1, w2, w3 = p['W1'], p['W2'], p['W3']
    b1, b2, b3 = (p[k].reshape(1, -1) for k in ('b1', 'b2', 'b3'))
    return pl.pallas_call(
        _node_mlp_body,
        grid=(_N // _NB,),
        in_specs=[
            pl.BlockSpec((_NB, _D), lambda i: (i, 0)),
            _full(w1.shape), _full(b1.shape),
            _full(w2.shape), _full(b2.shape),
            _full(w3.shape), _full(b3.shape),
        ],
        out_specs=pl.BlockSpec((_NB, _D), lambda i: (i, 0)),
        out_shape=jax.ShapeDtypeStruct((_N, _D), _F32),
    )(x, w1, b1, w2, b2, w3, b3)


_BF16 = jnp.bfloat16


def _edge_mlp_body(hs_ref, hd_ref, le_ref,
                   w1e, b1e, w2e, b2e,
                   wa, wb, w3c, b3c,
                   w21, b21, w20, b20, w31, b31, w30, b30,
                   m1_ref, m0_ref):
    le = le_ref[...]
    h1 = jnp.maximum(le * w1e[...] + b1e[...], 0.0)
    he2 = jnp.maximum(_dot(h1.astype(_BF16), w2e[...]) + b2e[...], 0.0)
    pre = (_dot(hs_ref[...].astype(_BF16), wa[...])
           + _dot(hd_ref[...].astype(_BF16), wb[...])
           + _dot(he2.astype(_BF16), w3c[...]) + b3c[...])
    t = jnp.maximum(pre, 0.0).astype(_BF16)
    u1 = jnp.maximum(_dot(t[:, :256], w21[...]) + b21[...], 0.0)
    u0 = jnp.maximum(_dot(t[:, 256:], w20[...]) + b20[...], 0.0)
    m1_ref[...] = _dot(u1.astype(_BF16), w31[...]) + b31[...]
    m0_ref[...] = _dot(u0.astype(_BF16), w30[...]) + b30[...]


def _edge_mlp(hs, hd, l_e, p_e, p_e1, p_e0):
    w1cat = jnp.concatenate([p_e1['W1'], p_e0['W1']], axis=1)
    wa, wb, wc = w1cat[:_D], w1cat[_D:2 * _D], w1cat[2 * _D:]
    b1cat = jnp.concatenate([p_e1['b1'], p_e0['b1']])
    w3c = p_e['W3'] @ wc
    b3c = (p_e['b3'] @ wc + b1cat).reshape(1, -1)
    w1e, b1e = p_e['W1'], p_e['b1'].reshape(1, -1)
    w2e, b2e = p_e['W2'].astype(_BF16), p_e['b2'].reshape(1, -1)
    w21, b21 = p_e1['W2'].astype(_BF16), p_e1['b2'].reshape(1, -1)
    w20, b20 = p_e0['W2'].astype(_BF16), p_e0['b2'].reshape(1, -1)
    w31, b31 = p_e1['W3'].astype(_BF16), p_e1['b3'].reshape(1, -1)
    w30, b30 = p_e0['W3'].astype(_BF16), p_e0['b3'].reshape(1, -1)
    wa, wb, w3c = wa.astype(_BF16), wb.astype(_BF16), w3c.astype(_BF16)
    args = (hs, hd, l_e, w1e, b1e, w2e, b2e, wa, wb, w3c, b3c,
            w21, b21, w20, b20, w31, b31, w30, b30)
    return pl.pallas_call(
        _edge_mlp_body,
        grid=(_E // _EB,),
        in_specs=[
            pl.BlockSpec((_EB, _D), lambda i: (i, 0)),
            pl.BlockSpec((_EB, _D), lambda i: (i, 0)),
            pl.BlockSpec((_EB, 1), lambda i: (i, 0)),
        ] + [_full(a.shape) for a in args[3:]],
        out_specs=[pl.BlockSpec((_EB, _D), lambda i: (i, 0))] * 2,
        out_shape=[jax.ShapeDtypeStruct((_E, _D), _F32)] * 2,
    )(*args)


def _final_body(hv_ref, s1_ref, s0_ref, deg0_ref, deg1_ref,
                wa, wb, wc, b1, w2, b2, w3, b3, o_ref):
    deg = deg0_ref[...][:, 0:1] + deg1_ref[...][:, 0:1]
    r = 1.0 / jnp.maximum(deg, 1.0)
    hv = hv_ref[...]
    t = jnp.maximum(_dot(hv, wa[...]) + _dot(s1_ref[...] * r, wb[...])
                    + _dot(s0_ref[...] * r, wc[...]) + b1[...], 0.0)
    u = jnp.maximum(_dot(t, w2[...]) + b2[...], 0.0)
    o_ref[...] = _dot(u, w3[...]) + b3[...] + hv


def _finalize(hv, s1, s0, deg0, deg1, p):
    wa, wb, wc = p['W1'][:_D], p['W1'][_D:2 * _D], p['W1'][2 * _D:]
    b1, b2, b3 = (p[k].reshape(1, -1) for k in ('b1', 'b2', 'b3'))
    w2, w3 = p['W2'], p['W3']
    args = (hv, s1, s0, deg0, deg1, wa, wb, wc, b1, w2, b2, w3, b3)
    return pl.pallas_call(
        _final_body,
        grid=(_N // _NB,),
        in_specs=[
            pl.BlockSpec((_NB, _D), lambda i: (i, 0)),
            pl.BlockSpec((_NB, _D), lambda i: (i, 0)),
            pl.BlockSpec((_NB, _D), lambda i: (i, 0)),
            pl.BlockSpec((_NB, _D), lambda i: (i, 0)),
            pl.BlockSpec((_NB, _D), lambda i: (i, 0)),
        ] + [_full(a.shape) for a in args[5:]],
        out_specs=pl.BlockSpec((_NB, _D), lambda i: (i, 0)),
        out_shape=jax.ShapeDtypeStruct((_N, _D), _F32),
    )(*args)


_NC, _NS = 2, 16
_NW = _NC * _NS
_GCH = 80
_IDXROWS = _E // _GCH
_EPW = _E // _NW
_GROWS = _EPW // _GCH
_EPT = _E // _NS
_SROWS = _EPT // _GCH
_NSEG = _N // 10

@functools.cache
def _sc_mesh():
    return plsc.VectorSubcoreMesh(core_axis_name="c", subcore_axis_name="s",
                                  num_cores=_NC, num_subcores=_NS)


def _gather_body(hv, src3, dst3, z128, ones128, out_s, out_d, deg0, deg1,
                 idx_v, rows_v, ones_v, degacc, sem):
    cid = lax.axis_index("c")
    sid = lax.axis_index("s")
    wid = sid * _NC + cid
    ebase = wid * _EPW

    @pl.when(sid < 10)
    def _():
        sl = pl.ds(sid * _NSEG, _NSEG)
        pltpu.sync_copy(z128.at[sl], degacc.at[sl])

    pltpu.sync_copy(ones128, ones_v)
    plsc.subcore_barrier()

    def run(idx3, out):
        pltpu.sync_copy(idx3.at[wid], idx_v)

        def go(j, carry):
            pltpu.async_copy(hv.at[idx_v.at[j]], rows_v, sem).wait()
            pltpu.sync_copy(rows_v, out.at[pl.ds(ebase + j * _GCH, _GCH)])
            return carry

        lax.fori_loop(0, _GROWS, go, 0)

    run(src3, out_s)
    run(dst3, out_d)

    def go_deg(j, carry):
        pltpu.sync_copy(ones_v, degacc.at[idx_v.at[j]], add=True)
        return carry

    lax.fori_loop(0, _GROWS, go_deg, 0)
    plsc.subcore_barrier()

    @pl.when((sid < 10) & (cid == 0))
    def _():
        sl = pl.ds(sid * _NSEG, _NSEG)
        pltpu.sync_copy(degacc.at[sl], deg0.at[sl])

    @pl.when((sid < 10) & (cid == 1))
    def _():
        sl = pl.ds(sid * _NSEG, _NSEG)
        pltpu.sync_copy(degacc.at[sl], deg1.at[sl])


@functools.cache
def _gather_kernel():
    return pl.kernel(
        _gather_body,
        out_type=[
            jax.ShapeDtypeStruct((_E, _D), _F32),
            jax.ShapeDtypeStruct((_E, _D), _F32),
            jax.ShapeDtypeStruct((_N, _D), _F32),
            jax.ShapeDtypeStruct((_N, _D), _F32),
        ],
        mesh=_sc_mesh(),
        scratch_types=[
            pltpu.VMEM((_GROWS, _GCH), jnp.int32),
            pltpu.VMEM((_GCH, _D), _F32),
            pltpu.VMEM((_GCH, _D), _F32),
            pltpu.VMEM_SHARED((_N, _D), _F32),
            pltpu.SemaphoreType.DMA,
        ],
    )


def _gather(hv, src3, dst3, z128, ones128):
    return _gather_kernel()(hv, src3, dst3, z128, ones128)


def _scatter_body(m1, m0, dst3s, z128,
                  out1, out0,
                  idx_v, rows_v, acc):
    cid = lax.axis_index("c")
    sid = lax.axis_index("s")

    @pl.when(sid < 10)
    def _():
        sl = pl.ds(sid * _NSEG, _NSEG)
        pltpu.sync_copy(z128.at[sl], acc.at[sl])

    ebase = sid * _EPT
    pltpu.sync_copy(dst3s.at[sid], idx_v)
    plsc.subcore_barrier()

    def accum(mref):
        def go(j, carry):
            pltpu.sync_copy(mref.at[pl.ds(ebase + j * _GCH, _GCH)], rows_v)
            pltpu.sync_copy(rows_v, acc.at[idx_v.at[j]], add=True)
            return carry
        lax.fori_loop(0, _SROWS, go, 0)

    @pl.when(cid == 0)
    def _():
        accum(m1)

    @pl.when(cid == 1)
    def _():
        accum(m0)

    plsc.subcore_barrier()

    @pl.when((sid < 10) & (cid == 0))
    def _():
        sl = pl.ds(sid * _NSEG, _NSEG)
        pltpu.sync_copy(acc.at[sl], out1.at[sl])

    @pl.when((sid < 10) & (cid == 1))
    def _():
        sl = pl.ds(sid * _NSEG, _NSEG)
        pltpu.sync_copy(acc.at[sl], out0.at[sl])


@functools.cache
def _scatter_kernel():
    return pl.kernel(
        _scatter_body,
        out_type=[
            jax.ShapeDtypeStruct((_N, _D), _F32),
            jax.ShapeDtypeStruct((_N, _D), _F32),
        ],
        mesh=_sc_mesh(),
        scratch_types=[
            pltpu.VMEM((_SROWS, _GCH), jnp.int32),
            pltpu.VMEM((_GCH, _D), _F32),
            pltpu.VMEM_SHARED((_N, _D), _F32),
        ],
    )


def _scatter(m1, m0, dst3s, z128):
    return _scatter_kernel()(m1, m0, dst3s, z128)


def kernel(x, l_e, edge_index, p_v, p_e, p_e1, p_e0, p_aggr):
    src3 = edge_index[0].reshape(_NW, _GROWS, _GCH)
    dst3 = edge_index[1].reshape(_NW, _GROWS, _GCH)
    dst3s = edge_index[1].reshape(_NS, _SROWS, _GCH)
    z128 = jnp.zeros((_N, _D), _F32)
    ones128 = jnp.ones((_GCH, _D), _F32)
    hv = _node_mlp(x, p_v)
    hs, hd, deg0, deg1 = _gather(hv, src3, dst3, z128, ones128)
    m1, m0 = _edge_mlp(hs, hd, l_e, p_e, p_e1, p_e0)
    s1, s0 = _scatter(m1, m0, dst3s, z128)
    return _finalize(hv, s1, s0, deg0, deg1, p_aggr)


# speedup vs baseline: 1.1271x; 1.1271x over previous
import functools

import jax
import jax.numpy as jnp
from jax import lax
from jax.experimental import pallas as pl
from jax.experimental.pallas import tpu as pltpu
from jax.experimental.pallas import tpu_sc as plsc

_N = 10000
_E = 320000
_D = 128

_F32 = jnp.float32


_NB = 1000
_EB = 2000


def _dot(a, b):
    return jnp.dot(a, b, preferred_element_type=_F32)


def _node_mlp_body(x_ref, w1, b1, w2, b2, w3, b3, o_ref):
    h = jnp.maximum(_dot(x_ref[...], w1[...]) + b1[...], 0.0)
    h = jnp.maximum(_dot(h, w2[...]) + b2[...], 0.0)
    o_ref[...] = _dot(h, w3[...]) + b3[...]


def _full(shape):
    return pl.BlockSpec(shape, lambda i: (0,) * len(shape))


def _node_mlp(x, p):
    w1, w2, w3 = p['W1'], p['W2'], p['W3']
    b1, b2, b3 = (p[k].reshape(1, -1) for k in ('b1', 'b2', 'b3'))
    return pl.pallas_call(
        _node_mlp_body,
        grid=(_N // _NB,),
        in_specs=[
            pl.BlockSpec((_NB, _D), lambda i: (i, 0)),
            _full(w1.shape), _full(b1.shape),
            _full(w2.shape), _full(b2.shape),
            _full(w3.shape), _full(b3.shape),
        ],
        out_specs=pl.BlockSpec((_NB, _D), lambda i: (i, 0)),
        out_shape=jax.ShapeDtypeStruct((_N, _D), _F32),
    )(x, w1, b1, w2, b2, w3, b3)


_BF16 = jnp.bfloat16


def _edge_mlp_body(hsd_ref, le_ref,
                   w1e, b1e, w2e, b2e,
                   wab, w3c, b3c,
                   w21, b21, w20, b20, w31, b31, w30, b30,
                   m1_ref, m0_ref):
    le = le_ref[...]
    h1 = jnp.maximum(le * w1e[...] + b1e[...], 0.0)
    he2 = jnp.maximum(_dot(h1.astype(_BF16), w2e[...]) + b2e[...], 0.0)
    pre = (_dot(hsd_ref[...].astype(_BF16), wab[...])
           + _dot(he2.astype(_BF16), w3c[...]) + b3c[...])
    t = jnp.maximum(pre, 0.0).astype(_BF16)
    u1 = jnp.maximum(_dot(t[:, :256], w21[...]) + b21[...], 0.0)
    u0 = jnp.maximum(_dot(t[:, 256:], w20[...]) + b20[...], 0.0)
    m1_ref[...] = _dot(u1.astype(_BF16), w31[...]) + b31[...]
    m0_ref[...] = _dot(u0.astype(_BF16), w30[...]) + b30[...]


def _edge_mlp(hsd, l_e, p_e, p_e1, p_e0):
    w1cat = jnp.concatenate([p_e1['W1'], p_e0['W1']], axis=1)
    wa, wb, wc = w1cat[:_D], w1cat[_D:2 * _D], w1cat[2 * _D:]
    b1cat = jnp.concatenate([p_e1['b1'], p_e0['b1']])
    w3c = p_e['W3'] @ wc
    b3c = (p_e['b3'] @ wc + b1cat).reshape(1, -1)
    w1e, b1e = p_e['W1'], p_e['b1'].reshape(1, -1)
    w2e, b2e = p_e['W2'].astype(_BF16), p_e['b2'].reshape(1, -1)
    w21, b21 = p_e1['W2'].astype(_BF16), p_e1['b2'].reshape(1, -1)
    w20, b20 = p_e0['W2'].astype(_BF16), p_e0['b2'].reshape(1, -1)
    w31, b31 = p_e1['W3'].astype(_BF16), p_e1['b3'].reshape(1, -1)
    w30, b30 = p_e0['W3'].astype(_BF16), p_e0['b3'].reshape(1, -1)
    wab = jnp.concatenate([wa, wb], axis=0).astype(_BF16)
    w3c = w3c.astype(_BF16)
    args = (hsd, l_e, w1e, b1e, w2e, b2e, wab, w3c, b3c,
            w21, b21, w20, b20, w31, b31, w30, b30)
    return pl.pallas_call(
        _edge_mlp_body,
        grid=(_E // _EB,),
        in_specs=[
            pl.BlockSpec((_EB, 2 * _D), lambda i: (i, 0)),
            pl.BlockSpec((_EB, 1), lambda i: (i, 0)),
        ] + [_full(a.shape) for a in args[2:]],
        out_specs=[pl.BlockSpec((_EB, _D), lambda i: (i, 0))] * 2,
        out_shape=[jax.ShapeDtypeStruct((_E, _D), _F32)] * 2,
    )(*args)


def _final_body(hv_ref, s1_ref, s0_ref, deg0_ref, deg1_ref,
                wa, wb, wc, b1, w2, b2, w3, b3, o_ref):
    deg = deg0_ref[...][:, 0:1] + deg1_ref[...][:, 0:1]
    r = 1.0 / jnp.maximum(deg, 1.0)
    hv = hv_ref[...]
    t = jnp.maximum(_dot(hv, wa[...]) + _dot(s1_ref[...] * r, wb[...])
                    + _dot(s0_ref[...] * r, wc[...]) + b1[...], 0.0)
    u = jnp.maximum(_dot(t, w2[...]) + b2[...], 0.0)
    o_ref[...] = _dot(u, w3[...]) + b3[...] + hv


def _finalize(hv, s1, s0, deg0, deg1, p):
    wa, wb, wc = p['W1'][:_D], p['W1'][_D:2 * _D], p['W1'][2 * _D:]
    b1, b2, b3 = (p[k].reshape(1, -1) for k in ('b1', 'b2', 'b3'))
    w2, w3 = p['W2'], p['W3']
    args = (hv, s1, s0, deg0, deg1, wa, wb, wc, b1, w2, b2, w3, b3)
    return pl.pallas_call(
        _final_body,
        grid=(_N // _NB,),
        in_specs=[
            pl.BlockSpec((_NB, _D), lambda i: (i, 0)),
            pl.BlockSpec((_NB, _D), lambda i: (i, 0)),
            pl.BlockSpec((_NB, _D), lambda i: (i, 0)),
            pl.BlockSpec((_NB, _D), lambda i: (i, 0)),
            pl.BlockSpec((_NB, _D), lambda i: (i, 0)),
        ] + [_full(a.shape) for a in args[5:]],
        out_specs=pl.BlockSpec((_NB, _D), lambda i: (i, 0)),
        out_shape=jax.ShapeDtypeStruct((_N, _D), _F32),
    )(*args)


_NC, _NS = 2, 16
_NW = _NC * _NS
_GCH = 80
_IDXROWS = _E // _GCH
_EPW = _E // _NW
_GROWS = _EPW // _GCH
_EPT = _E // _NS
_SROWS = _EPT // _GCH
_NSEG = _N // 10

@functools.cache
def _sc_mesh():
    return plsc.VectorSubcoreMesh(core_axis_name="c", subcore_axis_name="s",
                                  num_cores=_NC, num_subcores=_NS)


def _gather_body(hv, src3, dst3, z128, ones128, out_sd, deg0, deg1,
                 idx_v, rows_v, ones_v, degacc, sem):
    cid = lax.axis_index("c")
    sid = lax.axis_index("s")
    wid = sid * _NC + cid
    ebase = wid * _EPW

    @pl.when(sid < 10)
    def _():
        sl = pl.ds(sid * _NSEG, _NSEG)
        pltpu.sync_copy(z128.at[sl], degacc.at[sl])

    pltpu.sync_copy(ones128, ones_v)
    plsc.subcore_barrier()

    def run(idx3, col0):
        pltpu.sync_copy(idx3.at[wid], idx_v)

        def go(j, carry):
            pltpu.async_copy(hv.at[idx_v.at[j]], rows_v, sem).wait()
            pltpu.sync_copy(
                rows_v,
                out_sd.at[pl.ds(ebase + j * _GCH, _GCH), pl.ds(col0, _D)])
            return carry

        lax.fori_loop(0, _GROWS, go, 0)

    run(src3, 0)
    run(dst3, _D)

    def go_deg(j, carry):
        pltpu.sync_copy(ones_v, degacc.at[idx_v.at[j]], add=True)
        return carry

    lax.fori_loop(0, _GROWS, go_deg, 0)
    plsc.subcore_barrier()

    @pl.when((sid < 10) & (cid == 0))
    def _():
        sl = pl.ds(sid * _NSEG, _NSEG)
        pltpu.sync_copy(degacc.at[sl], deg0.at[sl])

    @pl.when((sid < 10) & (cid == 1))
    def _():
        sl = pl.ds(sid * _NSEG, _NSEG)
        pltpu.sync_copy(degacc.at[sl], deg1.at[sl])


@functools.cache
def _gather_kernel():
    return pl.kernel(
        _gather_body,
        out_type=[
            jax.ShapeDtypeStruct((_E, 2 * _D), _F32),
            jax.ShapeDtypeStruct((_N, _D), _F32),
            jax.ShapeDtypeStruct((_N, _D), _F32),
        ],
        mesh=_sc_mesh(),
        scratch_types=[
            pltpu.VMEM((_GROWS, _GCH), jnp.int32),
            pltpu.VMEM((_GCH, _D), _F32),
            pltpu.VMEM((_GCH, _D), _F32),
            pltpu.VMEM_SHARED((_N, _D), _F32),
            pltpu.SemaphoreType.DMA,
        ],
    )


def _gather(hv, src3, dst3, z128, ones128):
    return _gather_kernel()(hv, src3, dst3, z128, ones128)


def _scatter_body(m1, m0, dst3s, z128,
                  out1, out0,
                  idx_v, rows_v, acc):
    cid = lax.axis_index("c")
    sid = lax.axis_index("s")

    @pl.when(sid < 10)
    def _():
        sl = pl.ds(sid * _NSEG, _NSEG)
        pltpu.sync_copy(z128.at[sl], acc.at[sl])

    ebase = sid * _EPT
    pltpu.sync_copy(dst3s.at[sid], idx_v)
    plsc.subcore_barrier()

    def accum(mref):
        def go(j, carry):
            pltpu.sync_copy(mref.at[pl.ds(ebase + j * _GCH, _GCH)], rows_v)
            pltpu.sync_copy(rows_v, acc.at[idx_v.at[j]], add=True)
            return carry
        lax.fori_loop(0, _SROWS, go, 0)

    @pl.when(cid == 0)
    def _():
        accum(m1)

    @pl.when(cid == 1)
    def _():
        accum(m0)

    plsc.subcore_barrier()

    @pl.when((sid < 10) & (cid == 0))
    def _():
        sl = pl.ds(sid * _NSEG, _NSEG)
        pltpu.sync_copy(acc.at[sl], out1.at[sl])

    @pl.when((sid < 10) & (cid == 1))
    def _():
        sl = pl.ds(sid * _NSEG, _NSEG)
        pltpu.sync_copy(acc.at[sl], out0.at[sl])


@functools.cache
def _scatter_kernel():
    return pl.kernel(
        _scatter_body,
        out_type=[
            jax.ShapeDtypeStruct((_N, _D), _F32),
            jax.ShapeDtypeStruct((_N, _D), _F32),
        ],
        mesh=_sc_mesh(),
        scratch_types=[
            pltpu.VMEM((_SROWS, _GCH), jnp.int32),
            pltpu.VMEM((_GCH, _D), _F32),
            pltpu.VMEM_SHARED((_N, _D), _F32),
        ],
    )


def _scatter(m1, m0, dst3s, z128):
    return _scatter_kernel()(m1, m0, dst3s, z128)


def kernel(x, l_e, edge_index, p_v, p_e, p_e1, p_e0, p_aggr):
    src3 = edge_index[0].reshape(_NW, _GROWS, _GCH)
    dst3 = edge_index[1].reshape(_NW, _GROWS, _GCH)
    dst3s = edge_index[1].reshape(_NS, _SROWS, _GCH)
    z128 = jnp.zeros((_N, _D), _F32)
    ones128 = jnp.ones((_GCH, _D), _F32)
    hv = _node_mlp(x, p_v)
    hsd, deg0, deg1 = _gather(hv, src3, dst3, z128, ones128)
    m1, m0 = _edge_mlp(hsd, l_e, p_e, p_e1, p_e0)
    s1, s0 = _scatter(m1, m0, dst3s, z128)
    return _finalize(hv, s1, s0, deg0, deg1, p_aggr)


# speedup vs baseline: 1.1891x; 1.0549x over previous
import functools

import jax
import jax.numpy as jnp
from jax import lax
from jax.experimental import pallas as pl
from jax.experimental.pallas import tpu as pltpu
from jax.experimental.pallas import tpu_sc as plsc

_N = 10000
_E = 320000
_D = 128

_F32 = jnp.float32


_NB = 1000
_EB = 2000


def _dot(a, b):
    return jnp.dot(a, b, preferred_element_type=_F32)


def _node_mlp_body(x_ref, w1, b1, w2, b2, w3, b3, o_ref):
    h = jnp.maximum(_dot(x_ref[...], w1[...]) + b1[...], 0.0)
    h = jnp.maximum(_dot(h, w2[...]) + b2[...], 0.0)
    o_ref[...] = _dot(h, w3[...]) + b3[...]


def _full(shape):
    return pl.BlockSpec(shape, lambda i: (0,) * len(shape))


def _node_mlp(x, p):
    w1, w2, w3 = p['W1'], p['W2'], p['W3']
    b1, b2, b3 = (p[k].reshape(1, -1) for k in ('b1', 'b2', 'b3'))
    return pl.pallas_call(
        _node_mlp_body,
        grid=(_N // _NB,),
        in_specs=[
            pl.BlockSpec((_NB, _D), lambda i: (i, 0)),
            _full(w1.shape), _full(b1.shape),
            _full(w2.shape), _full(b2.shape),
            _full(w3.shape), _full(b3.shape),
        ],
        out_specs=pl.BlockSpec((_NB, _D), lambda i: (i, 0)),
        out_shape=jax.ShapeDtypeStruct((_N, _D), _F32),
    )(x, w1, b1, w2, b2, w3, b3)


_BF16 = jnp.bfloat16


def _edge_mlp_body(hsd_ref, le_ref,
                   w1e, b1e, w2e, b2e,
                   wab, w3c, b3c,
                   w21, b21, w20, b20, w31, b31, w30, b30,
                   m1_ref, m0_ref):
    le = le_ref[...]
    h1 = jnp.maximum(le * w1e[...] + b1e[...], 0.0)
    he2 = jnp.maximum(_dot(h1.astype(_BF16), w2e[...]) + b2e[...], 0.0)
    pre = (_dot(hsd_ref[...].astype(_BF16), wab[...])
           + _dot(he2.astype(_BF16), w3c[...]) + b3c[...])
    t = jnp.maximum(pre, 0.0).astype(_BF16)
    u1 = jnp.maximum(_dot(t[:, :256], w21[...]) + b21[...], 0.0)
    u0 = jnp.maximum(_dot(t[:, 256:], w20[...]) + b20[...], 0.0)
    m1_ref[...] = _dot(u1.astype(_BF16), w31[...]) + b31[...]
    m0_ref[...] = _dot(u0.astype(_BF16), w30[...]) + b30[...]


def _edge_mlp(hsd, l_e, p_e, p_e1, p_e0):
    w1cat = jnp.concatenate([p_e1['W1'], p_e0['W1']], axis=1)
    wa, wb, wc = w1cat[:_D], w1cat[_D:2 * _D], w1cat[2 * _D:]
    b1cat = jnp.concatenate([p_e1['b1'], p_e0['b1']])
    w3c = p_e['W3'] @ wc
    b3c = (p_e['b3'] @ wc + b1cat).reshape(1, -1)
    w1e, b1e = p_e['W1'], p_e['b1'].reshape(1, -1)
    w2e, b2e = p_e['W2'].astype(_BF16), p_e['b2'].reshape(1, -1)
    w21, b21 = p_e1['W2'].astype(_BF16), p_e1['b2'].reshape(1, -1)
    w20, b20 = p_e0['W2'].astype(_BF16), p_e0['b2'].reshape(1, -1)
    w31, b31 = p_e1['W3'].astype(_BF16), p_e1['b3'].reshape(1, -1)
    w30, b30 = p_e0['W3'].astype(_BF16), p_e0['b3'].reshape(1, -1)
    wab = jnp.concatenate([wa, wb], axis=0).astype(_BF16)
    w3c = w3c.astype(_BF16)
    args = (hsd, l_e, w1e, b1e, w2e, b2e, wab, w3c, b3c,
            w21, b21, w20, b20, w31, b31, w30, b30)
    return pl.pallas_call(
        _edge_mlp_body,
        grid=(_E // _EB,),
        in_specs=[
            pl.BlockSpec((_EB, 2 * _D), lambda i: (i, 0)),
            pl.BlockSpec((_EB, 1), lambda i: (i, 0)),
        ] + [_full(a.shape) for a in args[2:]],
        out_specs=[pl.BlockSpec((_EB, _D), lambda i: (i, 0))] * 2,
        out_shape=[jax.ShapeDtypeStruct((_E, _D), _F32)] * 2,
    )(*args)


def _final_body(hv_ref, s1_ref, s0_ref, deg0_ref, deg1_ref,
                wa, wb, wc, b1, w2, b2, w3, b3, o_ref):
    deg = deg0_ref[...][:, 0:1] + deg1_ref[...][:, 0:1]
    r = 1.0 / jnp.maximum(deg, 1.0)
    hv = hv_ref[...]
    t = jnp.maximum(_dot(hv, wa[...]) + _dot(s1_ref[...] * r, wb[...])
                    + _dot(s0_ref[...] * r, wc[...]) + b1[...], 0.0)
    u = jnp.maximum(_dot(t, w2[...]) + b2[...], 0.0)
    o_ref[...] = _dot(u, w3[...]) + b3[...] + hv


def _finalize(hv, s1, s0, deg0, deg1, p):
    wa, wb, wc = p['W1'][:_D], p['W1'][_D:2 * _D], p['W1'][2 * _D:]
    b1, b2, b3 = (p[k].reshape(1, -1) for k in ('b1', 'b2', 'b3'))
    w2, w3 = p['W2'], p['W3']
    args = (hv, s1, s0, deg0, deg1, wa, wb, wc, b1, w2, b2, w3, b3)
    return pl.pallas_call(
        _final_body,
        grid=(_N // _NB,),
        in_specs=[
            pl.BlockSpec((_NB, _D), lambda i: (i, 0)),
            pl.BlockSpec((_NB, _D), lambda i: (i, 0)),
            pl.BlockSpec((_NB, _D), lambda i: (i, 0)),
            pl.BlockSpec((_NB, _D), lambda i: (i, 0)),
            pl.BlockSpec((_NB, _D), lambda i: (i, 0)),
        ] + [_full(a.shape) for a in args[5:]],
        out_specs=pl.BlockSpec((_NB, _D), lambda i: (i, 0)),
        out_shape=jax.ShapeDtypeStruct((_N, _D), _F32),
    )(*args)


_NC, _NS = 2, 16
_NW = _NC * _NS
_GCH = 80
_IDXROWS = _E // _GCH
_EPW = _E // _NW
_GROWS = _EPW // _GCH
_EPT = _E // _NS
_SROWS = _EPT // _GCH
_NSEG = _N // 10

@functools.cache
def _sc_mesh():
    return plsc.VectorSubcoreMesh(core_axis_name="c", subcore_axis_name="s",
                                  num_cores=_NC, num_subcores=_NS)


def _gather_body(hv, src3, dst3, z128, ones128, out_sd, deg0, deg1,
                 idx_v, rows_a, rows_b, ones_v, degacc,
                 gs_a, gs_b, ws_a, ws_b, dsem):
    cid = lax.axis_index("c")
    sid = lax.axis_index("s")
    wid = sid * _NC + cid
    ebase = wid * _EPW

    @pl.when(sid < 10)
    def _():
        sl = pl.ds(sid * _NSEG, _NSEG)
        pltpu.sync_copy(z128.at[sl], degacc.at[sl])

    pltpu.sync_copy(ones128, ones_v)
    plsc.subcore_barrier()

    def run(idx3, col0):
        pltpu.sync_copy(idx3.at[wid], idx_v)

        def gsrc(j):
            return hv.at[idx_v.at[j]]

        def wdst(j):
            return out_sd.at[pl.ds(ebase + j * _GCH, _GCH), pl.ds(col0, _D)]

        def start_g(j, buf, sem):
            pltpu.async_copy(gsrc(j), buf, sem)

        def wait_g(j, buf, sem):
            pltpu.make_async_copy(gsrc(j), buf, sem).wait()

        def start_w(j, buf, sem):
            pltpu.async_copy(buf, wdst(j), sem)

        def wait_w(j, buf, sem):
            pltpu.make_async_copy(buf, wdst(j), sem).wait()

        start_g(0, rows_a, gs_a)

        def go(i, carry):
            c0 = 2 * i
            wait_g(c0, rows_a, gs_a)
            start_w(c0, rows_a, ws_a)
            start_g(c0 + 1, rows_b, gs_b)
            wait_w(c0, rows_a, ws_a)
            wait_g(c0 + 1, rows_b, gs_b)
            start_g(c0 + 2, rows_a, gs_a)
            start_w(c0 + 1, rows_b, ws_b)
            wait_w(c0 + 1, rows_b, ws_b)
            return carry

        lax.fori_loop(0, (_GROWS - 1) // 2, go, 0)
        last = _GROWS - 1
        wait_g(last, rows_a, gs_a)
        start_w(last, rows_a, ws_a)
        wait_w(last, rows_a, ws_a)

    run(src3, 0)
    run(dst3, _D)

    def go_deg(i, carry):
        for b in range(5):
            pltpu.async_copy(ones_v, degacc.at[idx_v.at[5 * i + b]], dsem,
                             add=True)
        for b in range(5):
            pltpu.make_async_copy(ones_v, degacc.at[idx_v.at[5 * i + b]],
                                  dsem).wait()
        return carry

    lax.fori_loop(0, _GROWS // 5, go_deg, 0)
    plsc.subcore_barrier()

    @pl.when((sid < 10) & (cid == 0))
    def _():
        sl = pl.ds(sid * _NSEG, _NSEG)
        pltpu.sync_copy(degacc.at[sl], deg0.at[sl])

    @pl.when((sid < 10) & (cid == 1))
    def _():
        sl = pl.ds(sid * _NSEG, _NSEG)
        pltpu.sync_copy(degacc.at[sl], deg1.at[sl])


@functools.cache
def _gather_kernel():
    return pl.kernel(
        _gather_body,
        out_type=[
            jax.ShapeDtypeStruct((_E, 2 * _D), _F32),
            jax.ShapeDtypeStruct((_N, _D), _F32),
            jax.ShapeDtypeStruct((_N, _D), _F32),
        ],
        mesh=_sc_mesh(),
        scratch_types=[
            pltpu.VMEM((_GROWS, _GCH), jnp.int32),
            pltpu.VMEM((_GCH, _D), _F32),
            pltpu.VMEM((_GCH, _D), _F32),
            pltpu.VMEM((_GCH, _D), _F32),
            pltpu.VMEM_SHARED((_N, _D), _F32),
            pltpu.SemaphoreType.DMA,
            pltpu.SemaphoreType.DMA,
            pltpu.SemaphoreType.DMA,
            pltpu.SemaphoreType.DMA,
            pltpu.SemaphoreType.DMA,
        ],
    )


def _gather(hv, src3, dst3, z128, ones128):
    return _gather_kernel()(hv, src3, dst3, z128, ones128)


def _scatter_body(m1, m0, dst3s, z128,
                  out1, out0,
                  idx_v, rows_v, acc):
    cid = lax.axis_index("c")
    sid = lax.axis_index("s")

    @pl.when(sid < 10)
    def _():
        sl = pl.ds(sid * _NSEG, _NSEG)
        pltpu.sync_copy(z128.at[sl], acc.at[sl])

    ebase = sid * _EPT
    pltpu.sync_copy(dst3s.at[sid], idx_v)
    plsc.subcore_barrier()

    def accum(mref):
        def go(j, carry):
            pltpu.sync_copy(mref.at[pl.ds(ebase + j * _GCH, _GCH)], rows_v)
            pltpu.sync_copy(rows_v, acc.at[idx_v.at[j]], add=True)
            return carry
        lax.fori_loop(0, _SROWS, go, 0)

    @pl.when(cid == 0)
    def _():
        accum(m1)

    @pl.when(cid == 1)
    def _():
        accum(m0)

    plsc.subcore_barrier()

    @pl.when((sid < 10) & (cid == 0))
    def _():
        sl = pl.ds(sid * _NSEG, _NSEG)
        pltpu.sync_copy(acc.at[sl], out1.at[sl])

    @pl.when((sid < 10) & (cid == 1))
    def _():
        sl = pl.ds(sid * _NSEG, _NSEG)
        pltpu.sync_copy(acc.at[sl], out0.at[sl])


@functools.cache
def _scatter_kernel():
    return pl.kernel(
        _scatter_body,
        out_type=[
            jax.ShapeDtypeStruct((_N, _D), _F32),
            jax.ShapeDtypeStruct((_N, _D), _F32),
        ],
        mesh=_sc_mesh(),
        scratch_types=[
            pltpu.VMEM((_SROWS, _GCH), jnp.int32),
            pltpu.VMEM((_GCH, _D), _F32),
            pltpu.VMEM_SHARED((_N, _D), _F32),
        ],
    )


def _scatter(m1, m0, dst3s, z128):
    return _scatter_kernel()(m1, m0, dst3s, z128)


def kernel(x, l_e, edge_index, p_v, p_e, p_e1, p_e0, p_aggr):
    src3 = edge_index[0].reshape(_NW, _GROWS, _GCH)
    dst3 = edge_index[1].reshape(_NW, _GROWS, _GCH)
    dst3s = edge_index[1].reshape(_NS, _SROWS, _GCH)
    z128 = jnp.zeros((_N, _D), _F32)
    ones128 = jnp.ones((_GCH, _D), _F32)
    hv = _node_mlp(x, p_v)
    hsd, deg0, deg1 = _gather(hv, src3, dst3, z128, ones128)
    m1, m0 = _edge_mlp(hsd, l_e, p_e, p_e1, p_e0)
    s1, s0 = _scatter(m1, m0, dst3s, z128)
    return _finalize(hv, s1, s0, deg0, deg1, p_aggr)
